# baseline (speedup 1.0000x reference)
import jax
import jax.numpy as jnp
from jax import lax
from jax.experimental import pallas as pl
from jax.experimental.pallas import tpu as pltpu
from jax.experimental.pallas import tpu_sc as plsc

_B = 4096
_L = 200
_H = 32
_N = _B * _L
_NW = 32
_BT = _B // 128
_OUT_ELEMS = _B * _L * _H
_L_STRIDE = 4 * _BT * 8 * 128
_HH_STRIDE = _BT * 8 * 128
_QUADS = _L // 4


def _gather_body(idx_hbm, table_hbm, out_hbm, idx_v, rows, rowsT,
                 si0, si1, si2, si3, sg0, sg1, so0, so1):
    sem_i = (si0, si1, si2, si3)
    sem_g = (sg0, sg1)
    sem_o = (so0, so1)
    wid = lax.axis_index("s") * 2 + lax.axis_index("c")
    ibase = wid * 128
    obase = wid * 1024
    i128 = lax.iota(jnp.int32, 16) * 128

    def idx_off(l):
        return pl.multiple_of(l * _B + ibase, 128)

    def fire_idx(l, q):
        pltpu.async_copy(idx_hbm.at[pl.ds(idx_off(l), 128)],
                         idx_v.at[q], sem_i[q])

    def wait_idx(l, q):
        pltpu.make_async_copy(idx_hbm.at[pl.ds(idx_off(l), 128)],
                              idx_v.at[q], sem_i[q]).wait()

    def fire_gather(q, p):
        pltpu.async_copy(table_hbm.at[idx_v.at[q]], rows.at[p], sem_g[p])

    def wait_gather(q, p):
        pltpu.make_async_copy(table_hbm.at[idx_v.at[q]], rows.at[p],
                              sem_g[p]).wait()

    def out_off(l, hh):
        return pl.multiple_of(l * _L_STRIDE + hh * _HH_STRIDE + obase, 1024)

    def fire_wb(l, p):
        for hh in range(4):
            pltpu.async_copy(rowsT.at[p, pl.ds(hh * 1024, 1024)],
                             out_hbm.at[pl.ds(out_off(l, hh), 1024)],
                             sem_o[p])

    def wait_wb(l, p):
        for hh in range(4):
            pltpu.make_async_copy(rowsT.at[p, pl.ds(hh * 1024, 1024)],
                                  out_hbm.at[pl.ds(out_off(l, hh), 1024)],
                                  sem_o[p]).wait()

    def transpose(p):
        for b in range(128):
            v1 = rows[p, b, pl.ds(0, 16)]
            v2 = rows[p, b, pl.ds(16, 16)]
            plsc.store_scatter(rowsT.at[p], [i128 + b], v1)
            plsc.store_scatter(rowsT.at[p], [i128 + (b + 2048)], v2)

    for q in range(4):
        fire_idx(q, q)
    wait_idx(0, 0)
    fire_gather(0, 0)

    def body(t, carry):
        for j in range(4):
            l = 4 * t + j
            p = j % 2
            if j >= 2:
                wait_wb(l - 2, p)
            else:

                @pl.when(t > 0)
                def _():
                    wait_wb(l - 2, p)
            wait_gather(j, p)

            @pl.when(t < _QUADS - 1)
            def _():
                fire_idx(l + 4, j)
            if j < 3:
                wait_idx(l + 1, j + 1)
                fire_gather(j + 1, 1 - p)
            else:

                @pl.when(t < _QUADS - 1)
                def _():
                    wait_idx(l + 1, 0)
                    fire_gather(0, 1 - p)
            transpose(p)
            fire_wb(l, p)
        return carry

    lax.fori_loop(0, _QUADS, body, 0)

    wait_wb(_L - 2, 0)
    wait_wb(_L - 1, 1)


def kernel(input_ids, table):
    idx_lb = jnp.transpose(input_ids).reshape(_N).astype(jnp.int32)
    mesh = plsc.VectorSubcoreMesh(core_axis_name="c", subcore_axis_name="s")
    f = pl.kernel(
        _gather_body,
        mesh=mesh,
        compiler_params=pltpu.CompilerParams(use_tc_tiling_on_sc=False,
                                             needs_layout_passes=False),
        out_type=jax.ShapeDtypeStruct((_OUT_ELEMS,), jnp.float32),
        scratch_types=[
            pltpu.VMEM((4, 128), jnp.int32),
            pltpu.VMEM((2, 128, _H), jnp.float32),
            pltpu.VMEM((2, 4096), jnp.float32),
        ] + [pltpu.SemaphoreType.DMA] * 8,
    )
    out_flat = f(idx_lb, table)
    t = out_flat.reshape(_L, 4, _BT, 8, 128)
    return t.transpose(2, 4, 0, 1, 3).reshape(_B, _L, _H)
